# initial kernel scaffold (unmeasured)
import jax
import jax.numpy as jnp
from jax import lax
from jax.experimental import pallas as pl
from jax.experimental.pallas import tpu as pltpu

N_DEV = 4
SQ = 2048
SKV = 2048
H_PER = 8
DH = 128
DM = 1024
DW = H_PER * DH
QC = 1024
BLK = 64
SCALE = 0.08838834764831843

_bf16 = jnp.bfloat16
_f32 = jnp.float32
_MESH = pl.DeviceIdType.MESH


def _body(x_ref, w_ref, k_ref, v_ref, out_ref,
          w_comm, kblk, vblk,
          w_send, w_recv, kv_sem):
    me = lax.axis_index("i")
    right = lax.rem(me + 1, N_DEV)
    left = lax.rem(me + N_DEV - 1, N_DEV)

    barrier = pltpu.get_barrier_semaphore()
    for nbr in (left, right):
        pl.semaphore_signal(barrier, inc=1, device_id=(nbr,),
                            device_id_type=_MESH)
    pl.semaphore_wait(barrier, 2)

    sends = []
    hop0 = pltpu.make_async_remote_copy(
        src_ref=w_ref, dst_ref=w_comm.at[0],
        send_sem=w_send.at[0], recv_sem=w_recv.at[0],
        device_id=(right,), device_id_type=_MESH)
    hop0.start()
    sends.append(hop0)

    qb = lax.broadcasted_iota(jnp.int32, (SQ // BLK, SKV), 0) + me * (SQ // BLK)
    kb = lax.broadcasted_iota(jnp.int32, (SQ // BLK, SKV), 1) // BLK
    keep = jnp.logical_or(jnp.logical_or(qb == kb, kb == 0),
                          lax.rem(qb + kb, 3) == 0)
    bias = jnp.where(keep, 0.0, -1e9).astype(_f32)

    out_ref[...] = jnp.zeros((SQ, DM), _f32)
    xv = x_ref[...]

    def compute(j, w_j):
        ck = pltpu.make_async_copy(
            k_ref.at[:, pl.ds(j * DW, DW)], kblk, kv_sem.at[0])
        cv = pltpu.make_async_copy(
            v_ref.at[:, pl.ds(j * DW, DW)], vblk, kv_sem.at[1])
        ck.start()
        cv.start()
        wq_j = w_j[:DM, :]
        wo_j = w_j[DM:, :]
        q = lax.dot_general(xv, wq_j, (((1,), (0,)), ((), ())),
                            preferred_element_type=_f32)
        q = (q * SCALE).astype(_bf16)
        ck.wait()
        cv.wait()
        for h in range(H_PER):
            k_h = kblk[:, h * DH:(h + 1) * DH]
            v_h = vblk[:, h * DH:(h + 1) * DH]
            for c in range(SQ // QC):
                qh = q[c * QC:(c + 1) * QC, h * DH:(h + 1) * DH]
                scores = lax.dot_general(
                    qh, k_h, (((1,), (1,)), ((), ())),
                    preferred_element_type=_f32)
                nb = QC // BLK
                scores = (scores.reshape(nb, BLK, SKV)
                          + bias[c * nb:(c + 1) * nb, None, :])
                e = jnp.exp(scores).astype(_bf16).reshape(QC, SKV)
                s = jnp.sum(e, axis=-1, keepdims=True, dtype=_f32)
                ctx = lax.dot_general(
                    e, v_h, (((1,), (0,)), ((), ())),
                    preferred_element_type=_f32)
                ctx = (ctx / s).astype(_bf16)
                wo_h = w_j[DM + h * DH:DM + (h + 1) * DH, :]
                out_ref[c * QC:(c + 1) * QC, :] += lax.dot_general(
                    ctx, wo_h, (((1,), (0,)), ((), ())),
                    preferred_element_type=_f32)

    compute(me, w_ref[...])

    for t in range(1, N_DEV):
        slot = t - 1
        rec = pltpu.make_async_remote_copy(
            src_ref=w_comm.at[slot], dst_ref=w_comm.at[slot],
            send_sem=w_send.at[slot], recv_sem=w_recv.at[slot],
            device_id=(left,), device_id_type=_MESH)
        rec.wait_recv()
        if t < N_DEV - 1:
            fwd = pltpu.make_async_remote_copy(
                src_ref=w_comm.at[slot], dst_ref=w_comm.at[slot + 1],
                send_sem=w_send.at[slot + 1], recv_sem=w_recv.at[slot + 1],
                device_id=(right,), device_id_type=_MESH)
            fwd.start()
            sends.append(fwd)
        j = lax.rem(me + N_DEV - t, N_DEV)
        compute(j, w_comm[slot])

    for r in sends:
        r.wait_send()


def kernel(x, Wq, K_ext, V_ext, Wo):
    xb = x.reshape(SQ, DM).astype(_bf16)
    w = jnp.concatenate([Wq.astype(_bf16), Wo.astype(_bf16)], axis=0)
    k2 = K_ext.reshape(SKV, N_DEV * DW).astype(_bf16)
    v2 = V_ext.reshape(SKV, N_DEV * DW).astype(_bf16)

    out = pl.pallas_call(
        _body,
        out_shape=jax.ShapeDtypeStruct((SQ, DM), _f32),
        in_specs=[
            pl.BlockSpec(memory_space=pltpu.MemorySpace.VMEM),
            pl.BlockSpec(memory_space=pltpu.MemorySpace.VMEM),
            pl.BlockSpec(memory_space=pltpu.MemorySpace.ANY),
            pl.BlockSpec(memory_space=pltpu.MemorySpace.ANY),
        ],
        out_specs=pl.BlockSpec(memory_space=pltpu.MemorySpace.VMEM),
        scratch_shapes=[
            pltpu.VMEM((3, 2 * DM, DW), _bf16),
            pltpu.VMEM((SKV, DW), _bf16),
            pltpu.VMEM((SKV, DW), _bf16),
            pltpu.SemaphoreType.DMA((3,)),
            pltpu.SemaphoreType.DMA((3,)),
            pltpu.SemaphoreType.DMA((2,)),
        ],
        compiler_params=pltpu.CompilerParams(collective_id=0),
    )(xb, w, k2, v2)
    return out.reshape(1, SQ, DM)


# baseline (device time: 347629 ns/iter reference)
import jax
import jax.numpy as jnp
from jax import lax
from jax.experimental import pallas as pl
from jax.experimental.pallas import tpu as pltpu

N_DEV = 4
SQ = 2048
SKV = 2048
H_PER = 8
DH = 128
DM = 1024
DW = H_PER * DH
QC = 512
BLK = 64
SCALE = 0.08838834764831843

_bf16 = jnp.bfloat16
_f32 = jnp.float32
_MESH = pl.DeviceIdType.MESH


def _body(x_ref, w_ref, k_ref, v_ref, out_ref,
          w_comm, kblk, vblk, bias_scr,
          w_send, w_recv, kv_sem):
    me = lax.axis_index("i")
    right = lax.rem(me + 1, N_DEV)
    left = lax.rem(me + N_DEV - 1, N_DEV)

    barrier = pltpu.get_barrier_semaphore()
    for nbr in (left, right):
        pl.semaphore_signal(barrier, inc=1, device_id=(nbr,),
                            device_id_type=_MESH)
    pl.semaphore_wait(barrier, 2)

    sends = []
    hop0 = pltpu.make_async_remote_copy(
        src_ref=w_ref, dst_ref=w_comm.at[0],
        send_sem=w_send.at[0], recv_sem=w_recv.at[0],
        device_id=(right,), device_id_type=_MESH)
    hop0.start()
    sends.append(hop0)

    qb = lax.broadcasted_iota(jnp.int32, (SQ // BLK, SKV), 0) + me * (SQ // BLK)
    kb = lax.broadcasted_iota(jnp.int32, (SQ // BLK, SKV), 1) // BLK
    keep = jnp.logical_or(jnp.logical_or(qb == kb, kb == 0),
                          lax.rem(qb + kb, 3) == 0)
    bias_scr[...] = jnp.where(keep, 0.0, -1e9).astype(_f32)

    out_ref[...] = jnp.zeros((SQ, DM), _f32)

    def compute(j, wref):
        ck = pltpu.make_async_copy(
            k_ref.at[:, pl.ds(j * DW, DW)], kblk, kv_sem.at[0])
        cv = pltpu.make_async_copy(
            v_ref.at[:, pl.ds(j * DW, DW)], vblk, kv_sem.at[1])
        ck.start()
        cv.start()
        wq_j = wref[:DM, :]
        ck.wait()
        cv.wait()
        nb = QC // BLK

        def chunk_body(c, carry):
            row0 = c * QC
            xc = x_ref[pl.ds(row0, QC), :]
            q_c = lax.dot_general(xc, wq_j, (((1,), (0,)), ((), ())),
                                  preferred_element_type=_f32)
            q_c = (q_c * SCALE).astype(_bf16)
            bias_c = bias_scr[pl.ds(c * nb, nb), :][:, None, :]
            acc = out_ref[pl.ds(row0, QC), :]
            for h in range(H_PER):
                k_h = kblk[:, h * DH:(h + 1) * DH]
                v_h = vblk[:, h * DH:(h + 1) * DH]
                wo_h = wref[DM + h * DH:DM + (h + 1) * DH, :]
                qh = q_c[:, h * DH:(h + 1) * DH]
                scores = lax.dot_general(
                    qh, k_h, (((1,), (1,)), ((), ())),
                    preferred_element_type=_f32)
                scores = scores.reshape(nb, BLK, SKV) + bias_c
                e = jnp.exp(scores).astype(_bf16).reshape(QC, SKV)
                s = jnp.sum(e, axis=-1, keepdims=True, dtype=_f32)
                ctx = lax.dot_general(
                    e, v_h, (((1,), (0,)), ((), ())),
                    preferred_element_type=_f32)
                ctx = (ctx / s).astype(_bf16)
                acc = acc + lax.dot_general(
                    ctx, wo_h, (((1,), (0,)), ((), ())),
                    preferred_element_type=_f32)
            out_ref[pl.ds(row0, QC), :] = acc
            return carry

        lax.fori_loop(0, SQ // QC, chunk_body, 0)

    compute(me, w_ref)

    for t in range(1, N_DEV):
        slot = t - 1
        rec = pltpu.make_async_remote_copy(
            src_ref=w_comm.at[slot], dst_ref=w_comm.at[slot],
            send_sem=w_send.at[slot], recv_sem=w_recv.at[slot],
            device_id=(left,), device_id_type=_MESH)
        rec.wait_recv()
        if t < N_DEV - 1:
            fwd = pltpu.make_async_remote_copy(
                src_ref=w_comm.at[slot], dst_ref=w_comm.at[slot + 1],
                send_sem=w_send.at[slot + 1], recv_sem=w_recv.at[slot + 1],
                device_id=(right,), device_id_type=_MESH)
            fwd.start()
            sends.append(fwd)
        j = lax.rem(me + N_DEV - t, N_DEV)
        compute(j, w_comm.at[slot])

    for r in sends:
        r.wait_send()


def kernel(x, Wq, K_ext, V_ext, Wo):
    xb = x.reshape(SQ, DM).astype(_bf16)
    w = jnp.concatenate([Wq.astype(_bf16), Wo.astype(_bf16)], axis=0)
    k2 = K_ext.reshape(SKV, N_DEV * DW).astype(_bf16)
    v2 = V_ext.reshape(SKV, N_DEV * DW).astype(_bf16)

    out = pl.pallas_call(
        _body,
        out_shape=jax.ShapeDtypeStruct((SQ, DM), _f32),
        in_specs=[
            pl.BlockSpec(memory_space=pltpu.MemorySpace.VMEM),
            pl.BlockSpec(memory_space=pltpu.MemorySpace.VMEM),
            pl.BlockSpec(memory_space=pl.ANY),
            pl.BlockSpec(memory_space=pl.ANY),
        ],
        out_specs=pl.BlockSpec(memory_space=pltpu.MemorySpace.VMEM),
        scratch_shapes=[
            pltpu.VMEM((3, 2 * DM, DW), _bf16),
            pltpu.VMEM((SKV, DW), _bf16),
            pltpu.VMEM((SKV, DW), _bf16),
            pltpu.VMEM((SQ // BLK, SKV), _f32),
            pltpu.SemaphoreType.DMA((3,)),
            pltpu.SemaphoreType.DMA((3,)),
            pltpu.SemaphoreType.DMA((2,)),
        ],
        compiler_params=pltpu.CompilerParams(
            collective_id=0,
            vmem_limit_bytes=27 * 1024 * 1024,
        ),
    )(xb, w, k2, v2)
    return out.reshape(1, SQ, DM)
